# baseline (speedup 1.0000x reference)
import jax
import jax.numpy as jnp
from jax import lax
from jax.experimental import pallas as pl
from jax.experimental.pallas import tpu as pltpu
from jax.experimental.pallas import tpu_sc as plsc

BATCH = 4096
SEQ = 50
DIM = 64
LANES = 16

NUM_CORES = 2
NUM_SUBCORES = 16
NW = NUM_CORES * NUM_SUBCORES
SEQ_PER_W = BATCH // NW
CHUNK_SEQ = 8
ROWS_PER_CHUNK = CHUNK_SEQ * SEQ
SUB = 4
ROWS_PER_SUB = ROWS_PER_CHUNK // SUB
N_CHUNKS = SEQ_PER_W // CHUNK_SEQ
XROWS = BATCH * SEQ // ROWS_PER_SUB
NBUF = 3


SEQ_PAD = 64


def _sc_body(x_hbm, tab_hbm, out_hbm, idx_v, rows_v, sem_g, sem_o):
    wid = lax.axis_index("s") * NUM_CORES + lax.axis_index("c")
    xrow_base = wid * (SEQ_PER_W * SEQ // ROWS_PER_SUB)

    def issue(c):
        b = c % NBUF
        xrow = xrow_base + c * SUB
        pltpu.sync_copy(x_hbm.at[pl.ds(xrow, SUB)], idx_v.at[b])
        return [pltpu.async_copy(tab_hbm.at[idx_v.at[b].at[j]],
                                 rows_v.at[b].at[j], sem_g[b])
                for j in range(SUB)]

    def store(c):
        b = c % NBUF
        s0 = wid * SEQ_PER_W + c * CHUNK_SEQ
        cps = []
        for q in range(CHUNK_SEQ):
            for h in range(2):
                cps.append(pltpu.async_copy(
                    rows_v.at[b, q // 2,
                              pl.ds((q % 2) * SEQ + h * (SEQ // 2),
                                    SEQ // 2)],
                    out_hbm.at[s0 + q, pl.ds(0, SEQ // 2),
                               pl.ds(h * DIM, DIM)],
                    sem_o[b]))
        return cps

    gcps = [None] * N_CHUNKS
    ocps = [None] * N_CHUNKS
    gcps[0] = issue(0)
    for c in range(1, N_CHUNKS + 1):
        if c < N_CHUNKS:
            if c >= NBUF:
                for cp in ocps[c - NBUF]:
                    cp.wait()
            gcps[c] = issue(c)
        for cp in gcps[c - 1]:
            cp.wait()
        ocps[c - 1] = store(c - 1)
    for c in range(N_CHUNKS - NBUF, N_CHUNKS):
        for cp in ocps[c]:
            cp.wait()


TG = 16
NTG = SEQ_PAD // TG


def _tc_body(lin_ref, len_ref, o5_ref, r4_ref):
    j = pl.program_id(1)
    x = lin_ref[...]
    z = jnp.transpose(x, (1, 2, 0))
    o5_ref[...] = z.reshape(TG, 8, 1, 8, 128)
    riota = lax.broadcasted_iota(jnp.int32, (128, TG // 2, 128), 1)
    piota = lax.broadcasted_iota(jnp.int32, (128, TG // 2, 128), 2) // DIM
    tmask = (j * TG + 2 * riota + piota) < SEQ
    s2 = jnp.sum(jnp.where(tmask, x, 0.0), axis=1)
    part = s2[:, :DIM] + s2[:, DIM:]
    pt = jnp.transpose(part, (1, 0)).reshape(8, 1, 8, 128)

    @pl.when(j == 0)
    def _():
        r4_ref[...] = pt

    @pl.when(j > 0)
    def _():
        r4_ref[...] = r4_ref[...] + pt

    @pl.when(j == NTG - 1)
    def _():
        recip = 1.0 / len_ref[...].astype(jnp.float32)
        r4_ref[...] = r4_ref[...] * recip.reshape(1, 1, 1, 128)


@jax.jit
def _run(x2, x_len, emb_weight):
    mesh = plsc.VectorSubcoreMesh(core_axis_name="c", subcore_axis_name="s")
    sck = pl.kernel(
        _sc_body,
        mesh=mesh,
        compiler_params=pltpu.CompilerParams(
            needs_layout_passes=False, use_tc_tiling_on_sc=False),
        out_type=(
            jax.ShapeDtypeStruct((BATCH, SEQ_PAD * DIM // 128, 128),
                                 jnp.float32),
        ),
        scratch_types=[
            pltpu.VMEM((NBUF, SUB, ROWS_PER_SUB), jnp.int32),
            pltpu.VMEM((NBUF, SUB, ROWS_PER_SUB, DIM), jnp.float32),
            [pltpu.SemaphoreType.DMA] * NBUF,
            [pltpu.SemaphoreType.DMA] * NBUF,
        ],
    )
    (lin,) = sck(x2, emb_weight)

    o5, r4 = pl.pallas_call(
        _tc_body,
        grid=(NW, NTG),
        in_specs=[
            pl.BlockSpec((SEQ_PER_W, TG * DIM // 128, 128),
                         lambda i, j: (i, j, 0)),
            pl.BlockSpec((SEQ_PER_W,), lambda i, j: (i,)),
        ],
        out_specs=[
            pl.BlockSpec((TG, 8, 1, 8, 128), lambda i, j: (j, 0, i, 0, 0)),
            pl.BlockSpec((8, 1, 8, 128), lambda i, j: (0, i, 0, 0)),
        ],
        out_shape=(
            jax.ShapeDtypeStruct((SEQ, 8, NW, 8, 128), jnp.float32),
            jax.ShapeDtypeStruct((8, NW, 8, 128), jnp.float32),
        ),
    )(lin, x_len)
    return o5, r4


def kernel(x, x_len, emb_weight):
    perm = jnp.concatenate([jnp.arange(0, SEQ, 2), jnp.arange(1, SEQ, 2)])
    x2 = x[:, perm].astype(jnp.int32).reshape(XROWS, ROWS_PER_SUB)
    o5, r4 = _run(x2, x_len.astype(jnp.int32), emb_weight)
    out = o5.transpose((2, 4, 0, 1, 3)).reshape(BATCH, SEQ, DIM)
    ret = r4.transpose((1, 3, 0, 2)).reshape(BATCH, DIM)
    return (ret, out)


# speedup vs baseline: 3.0170x; 3.0170x over previous
import jax
import jax.numpy as jnp
from jax import lax
from jax.experimental import pallas as pl
from jax.experimental.pallas import tpu as pltpu
from jax.experimental.pallas import tpu_sc as plsc

BATCH = 4096
SEQ = 50
DIM = 64
LANES = 16

NUM_CORES = 2
NUM_SUBCORES = 16
NW = NUM_CORES * NUM_SUBCORES
SEQ_PER_W = BATCH // NW
CHUNK_SEQ = 8
ROWS_PER_CHUNK = CHUNK_SEQ * SEQ
SUB = 4
ROWS_PER_SUB = ROWS_PER_CHUNK // SUB
N_CHUNKS = SEQ_PER_W // CHUNK_SEQ
XROWS = BATCH * SEQ // ROWS_PER_SUB
NBUF = 3


SEQ_PAD = 64


def _sc_body(x_hbm, tab_hbm, out_hbm, idx_v, rows_v, sem_g, sem_o):
    wid = lax.axis_index("s") * NUM_CORES + lax.axis_index("c")
    xrow_base = wid * (SEQ_PER_W * SEQ // ROWS_PER_SUB)

    def issue(c):
        b = c % NBUF
        xrow = xrow_base + c * SUB
        pltpu.sync_copy(x_hbm.at[pl.ds(xrow, SUB)], idx_v.at[b])
        return [pltpu.async_copy(tab_hbm.at[idx_v.at[b].at[j]],
                                 rows_v.at[b].at[j], sem_g[b])
                for j in range(SUB)]

    def store(c):
        b = c % NBUF
        s0 = wid * SEQ_PER_W + c * CHUNK_SEQ
        cps = []
        for q in range(CHUNK_SEQ):
            for h in range(2):
                cps.append(pltpu.async_copy(
                    rows_v.at[b, q // 2,
                              pl.ds((q % 2) * SEQ + h * (SEQ // 2),
                                    SEQ // 2)],
                    out_hbm.at[s0 + q, pl.ds(0, SEQ // 2),
                               pl.ds(h * DIM, DIM)],
                    sem_o[b]))
        return cps

    gcps = [None] * N_CHUNKS
    ocps = [None] * N_CHUNKS
    gcps[0] = issue(0)
    for c in range(1, N_CHUNKS + 1):
        if c < N_CHUNKS:
            if c >= NBUF:
                for cp in ocps[c - NBUF]:
                    cp.wait()
            gcps[c] = issue(c)
        for cp in gcps[c - 1]:
            cp.wait()
        ocps[c - 1] = store(c - 1)
    for c in range(N_CHUNKS - NBUF, N_CHUNKS):
        for cp in ocps[c]:
            cp.wait()


TG = 16
NTG = SEQ_PAD // TG


def _tc_body(lin_ref, len_ref, o5_ref, r4_ref):
    j = pl.program_id(1)
    s = jnp.zeros((128, 128), jnp.float32)
    for r in range(TG // 2):
        xr = lin_ref[:, r, :]
        o5_ref[pl.ds(2 * r, 2), :, 0, :, :] = (
            jnp.transpose(xr, (1, 0)).reshape(2, 8, 8, 128))
        ci = lax.broadcasted_iota(jnp.int32, (128, 128), 1)
        tmask = (j * TG + 2 * r + ci // DIM) < SEQ
        s = s + jnp.where(tmask, xr, 0.0)
    part = s[:, :DIM] + s[:, DIM:]
    pt = jnp.transpose(part, (1, 0)).reshape(8, 1, 8, 128)

    @pl.when(j == 0)
    def _():
        r4_ref[...] = pt

    @pl.when(j > 0)
    def _():
        r4_ref[...] = r4_ref[...] + pt

    @pl.when(j == NTG - 1)
    def _():
        recip = 1.0 / len_ref[...].astype(jnp.float32)
        r4_ref[...] = r4_ref[...] * recip.reshape(1, 1, 1, 128)


@jax.jit
def _run(x2, x_len, emb_weight):
    mesh = plsc.VectorSubcoreMesh(core_axis_name="c", subcore_axis_name="s")
    sck = pl.kernel(
        _sc_body,
        mesh=mesh,
        compiler_params=pltpu.CompilerParams(
            needs_layout_passes=False, use_tc_tiling_on_sc=False),
        out_type=(
            jax.ShapeDtypeStruct((BATCH, SEQ_PAD * DIM // 128, 128),
                                 jnp.float32),
        ),
        scratch_types=[
            pltpu.VMEM((NBUF, SUB, ROWS_PER_SUB), jnp.int32),
            pltpu.VMEM((NBUF, SUB, ROWS_PER_SUB, DIM), jnp.float32),
            [pltpu.SemaphoreType.DMA] * NBUF,
            [pltpu.SemaphoreType.DMA] * NBUF,
        ],
    )
    (lin,) = sck(x2, emb_weight)

    o5, r4 = pl.pallas_call(
        _tc_body,
        grid=(NW, NTG),
        in_specs=[
            pl.BlockSpec((SEQ_PER_W, TG * DIM // 128, 128),
                         lambda i, j: (i, j, 0)),
            pl.BlockSpec((SEQ_PER_W,), lambda i, j: (i,)),
        ],
        out_specs=[
            pl.BlockSpec((TG, 8, 1, 8, 128), lambda i, j: (j, 0, i, 0, 0)),
            pl.BlockSpec((8, 1, 8, 128), lambda i, j: (0, i, 0, 0)),
        ],
        out_shape=(
            jax.ShapeDtypeStruct((SEQ, 8, NW, 8, 128), jnp.float32),
            jax.ShapeDtypeStruct((8, NW, 8, 128), jnp.float32),
        ),
    )(lin, x_len)
    return o5, r4


def kernel(x, x_len, emb_weight):
    perm = jnp.concatenate([jnp.arange(0, SEQ, 2), jnp.arange(1, SEQ, 2)])
    x2 = x[:, perm].astype(jnp.int32).reshape(XROWS, ROWS_PER_SUB)
    o5, r4 = _run(x2, x_len.astype(jnp.int32), emb_weight)
    out = o5.transpose((2, 4, 0, 1, 3)).reshape(BATCH, SEQ, DIM)
    ret = r4.transpose((1, 3, 0, 2)).reshape(BATCH, DIM)
    return (ret, out)


# speedup vs baseline: 3.1612x; 1.0478x over previous
import jax
import jax.numpy as jnp
from jax import lax
from jax.experimental import pallas as pl
from jax.experimental.pallas import tpu as pltpu
from jax.experimental.pallas import tpu_sc as plsc

BATCH = 4096
SEQ = 50
DIM = 64
LANES = 16

NUM_CORES = 2
NUM_SUBCORES = 16
NW = NUM_CORES * NUM_SUBCORES
SEQ_PER_W = BATCH // NW
CHUNK_SEQ = 8
ROWS_PER_CHUNK = CHUNK_SEQ * SEQ
SUB = 4
ROWS_PER_SUB = ROWS_PER_CHUNK // SUB
XROWS = BATCH * SEQ // ROWS_PER_SUB
NBUF = 3
SEQ_PAD = 64

NH = 4
BQ = BATCH // NH
SEQ_PER_WQ = BQ // NW
N_CHUNKS_Q = SEQ_PER_WQ // CHUNK_SEQ
XROWS_Q = XROWS // NH

TG = 16
NTG = SEQ_PAD // TG
NWQ = NW // NH


def _make_sc_body(h):
    def _sc_body(x_hbm, tab_hbm, out_hbm, idx_v, rows_v, sem_g, sem_o):
        wid = lax.axis_index("s") * NUM_CORES + lax.axis_index("c")
        xrow_base = h * XROWS_Q + wid * (XROWS_Q // NW)

        def issue(c):
            b = c % NBUF
            xrow = xrow_base + c * SUB
            pltpu.sync_copy(x_hbm.at[pl.ds(xrow, SUB)], idx_v.at[b])
            return [pltpu.async_copy(tab_hbm.at[idx_v.at[b].at[j]],
                                     rows_v.at[b].at[j], sem_g[b])
                    for j in range(SUB)]

        def store(c):
            b = c % NBUF
            s0 = wid * SEQ_PER_WQ + c * CHUNK_SEQ
            cps = []
            for q in range(CHUNK_SEQ):
                for half in range(2):
                    cps.append(pltpu.async_copy(
                        rows_v.at[b, q // 2,
                                  pl.ds((q % 2) * SEQ + half * (SEQ // 2),
                                        SEQ // 2)],
                        out_hbm.at[s0 + q, pl.ds(0, SEQ // 2),
                                   pl.ds(half * DIM, DIM)],
                        sem_o[b]))
            return cps

        gcps = [None] * N_CHUNKS_Q
        ocps = [None] * N_CHUNKS_Q
        gcps[0] = issue(0)
        for c in range(1, N_CHUNKS_Q + 1):
            if c < N_CHUNKS_Q:
                if c >= NBUF:
                    for cp in ocps[c - NBUF]:
                        cp.wait()
                gcps[c] = issue(c)
            for cp in gcps[c - 1]:
                cp.wait()
            ocps[c - 1] = store(c - 1)
        for c in range(max(0, N_CHUNKS_Q - NBUF), N_CHUNKS_Q):
            for cp in ocps[c]:
                cp.wait()

    return _sc_body


def _make_tc_body(h, aliased):
    def body(*refs):
        if aliased:
            _, _, lin_ref, len_ref, o5_ref, r4_ref = refs
        else:
            lin_ref, len_ref, o5_ref, r4_ref = refs
        j = pl.program_id(1)
        s = jnp.zeros((128, 128), jnp.float32)
        for r in range(TG // 2):
            xr = lin_ref[:, r, :]
            o5_ref[pl.ds(2 * r, 2), :, 0, :, :] = (
                jnp.transpose(xr, (1, 0)).reshape(2, 8, 8, 128))
            ci = lax.broadcasted_iota(jnp.int32, (128, 128), 1)
            tmask = (j * TG + 2 * r + ci // DIM) < SEQ
            s = s + jnp.where(tmask, xr, 0.0)
        part = s[:, :DIM] + s[:, DIM:]
        pt = jnp.transpose(part, (1, 0)).reshape(8, 1, 8, 128)

        @pl.when(j == 0)
        def _():
            r4_ref[...] = pt

        @pl.when(j > 0)
        def _():
            r4_ref[...] = r4_ref[...] + pt

        @pl.when(j == NTG - 1)
        def _():
            recip = 1.0 / len_ref[...].astype(jnp.float32)
            r4_ref[...] = r4_ref[...] * recip.reshape(1, 1, 1, 128)

    return body


def _run_quarter_sc(h, x2, emb_weight):
    mesh = plsc.VectorSubcoreMesh(core_axis_name="c", subcore_axis_name="s")
    sck = pl.kernel(
        _make_sc_body(h),
        mesh=mesh,
        compiler_params=pltpu.CompilerParams(
            needs_layout_passes=False, use_tc_tiling_on_sc=False),
        out_type=(
            jax.ShapeDtypeStruct((BQ, SEQ_PAD * DIM // 128, 128),
                                 jnp.float32),
        ),
        scratch_types=[
            pltpu.VMEM((NBUF, SUB, ROWS_PER_SUB), jnp.int32),
            pltpu.VMEM((NBUF, SUB, ROWS_PER_SUB, DIM), jnp.float32),
            [pltpu.SemaphoreType.DMA] * NBUF,
            [pltpu.SemaphoreType.DMA] * NBUF,
        ],
    )
    (lin,) = sck(x2, emb_weight)
    return lin


def _run_quarter_tc(h, lin, len_h, o5_prev, r4_prev):
    aliased = o5_prev is not None
    out_shape = (
        jax.ShapeDtypeStruct((SEQ, 8, NW, 8, 128), jnp.float32),
        jax.ShapeDtypeStruct((8, NW, 8, 128), jnp.float32),
    )
    data_specs = [
        pl.BlockSpec((BQ // NWQ, TG * DIM // 128, 128),
                     lambda i, j: (i, j, 0)),
        pl.BlockSpec((BQ // NWQ,), lambda i, j: (i,)),
    ]
    out_specs = [
        pl.BlockSpec((TG, 8, 1, 8, 128),
                     lambda i, j, h=h: (j, 0, i + h * NWQ, 0, 0)),
        pl.BlockSpec((8, 1, 8, 128),
                     lambda i, j, h=h: (0, i + h * NWQ, 0, 0)),
    ]
    if aliased:
        in_specs = [pl.BlockSpec(memory_space=pl.ANY),
                    pl.BlockSpec(memory_space=pl.ANY)] + data_specs
        args = (o5_prev, r4_prev, lin, len_h)
        aliases = {0: 0, 1: 1}
    else:
        in_specs = data_specs
        args = (lin, len_h)
        aliases = {}
    return pl.pallas_call(
        _make_tc_body(h, aliased),
        grid=(NWQ, NTG),
        in_specs=in_specs,
        out_specs=out_specs,
        out_shape=out_shape,
        input_output_aliases=aliases,
    )(*args)


@jax.jit
def _run(x2, x_len, emb_weight):
    o5 = r4 = None
    lens = x_len.reshape(NH, BQ)
    for h in range(NH):
        lin = _run_quarter_sc(h, x2, emb_weight)
        o5, r4 = _run_quarter_tc(h, lin, lens[h], o5, r4)
    return o5, r4


def kernel(x, x_len, emb_weight):
    perm = jnp.concatenate([jnp.arange(0, SEQ, 2), jnp.arange(1, SEQ, 2)])
    x2 = x[:, perm].astype(jnp.int32).reshape(XROWS, ROWS_PER_SUB)
    o5, r4 = _run(x2, x_len.astype(jnp.int32), emb_weight)
    out = o5.transpose((2, 4, 0, 1, 3)).reshape(BATCH, SEQ, DIM)
    ret = r4.transpose((1, 3, 0, 2)).reshape(BATCH, DIM)
    return (ret, out)


# speedup vs baseline: 4.1288x; 1.3061x over previous
import jax
import jax.numpy as jnp
from jax import lax
from jax.experimental import pallas as pl
from jax.experimental.pallas import tpu as pltpu
from jax.experimental.pallas import tpu_sc as plsc

BATCH = 4096
SEQ = 50
DIM = 64
LANES = 16

NUM_CORES = 2
NUM_SUBCORES = 16
NW = NUM_CORES * NUM_SUBCORES
SEQ_PER_W = BATCH // NW
CHUNK_SEQ = 8
ROWS_PER_CHUNK = CHUNK_SEQ * SEQ
SUB = 4
ROWS_PER_SUB = ROWS_PER_CHUNK // SUB
XROWS = BATCH * SEQ // ROWS_PER_SUB
NBUF = 3
SEQ_PAD = 64

NH = 4
BQ = BATCH // NH
SEQ_PER_WQ = BQ // NW
N_CHUNKS_Q = SEQ_PER_WQ // CHUNK_SEQ
XROWS_Q = XROWS // NH

TG = 16
NTG = SEQ_PAD // TG
NWQ = NW // NH


def _make_sc_body(h):
    def _sc_body(x_hbm, tab_hbm, out_hbm, idx_v, rows_v, sem_g, sem_o):
        wid = lax.axis_index("s") * NUM_CORES + lax.axis_index("c")
        xrow_base = h * XROWS_Q + wid * (XROWS_Q // NW)

        def issue(c):
            b = c % NBUF
            xrow = xrow_base + c * SUB
            pltpu.sync_copy(x_hbm.at[pl.ds(xrow, SUB)], idx_v.at[b])
            return [pltpu.async_copy(tab_hbm.at[idx_v.at[b].at[j]],
                                     rows_v.at[b].at[j], sem_g[b])
                    for j in range(SUB)]

        def store(c):
            b = c % NBUF
            s0 = wid * SEQ_PER_WQ + c * CHUNK_SEQ
            cps = []
            for q in range(CHUNK_SEQ):
                for half in range(2):
                    cps.append(pltpu.async_copy(
                        rows_v.at[b, q // 2,
                                  pl.ds((q % 2) * SEQ + half * (SEQ // 2),
                                        SEQ // 2)],
                        out_hbm.at[s0 + q, pl.ds(0, SEQ // 2),
                                   pl.ds(half * DIM, DIM)],
                        sem_o[b]))
            return cps

        gcps = [None] * N_CHUNKS_Q
        ocps = [None] * N_CHUNKS_Q
        gcps[0] = issue(0)
        for c in range(1, N_CHUNKS_Q + 1):
            if c < N_CHUNKS_Q:
                if c >= NBUF:
                    for cp in ocps[c - NBUF]:
                        cp.wait()
                gcps[c] = issue(c)
            for cp in gcps[c - 1]:
                cp.wait()
            ocps[c - 1] = store(c - 1)
        for c in range(max(0, N_CHUNKS_Q - NBUF), N_CHUNKS_Q):
            for cp in ocps[c]:
                cp.wait()

    return _sc_body


def _make_tc_body(h, aliased):
    def body(*refs):
        if aliased:
            _, _, lin_ref, len_ref, o5_ref, r4_ref = refs
        else:
            lin_ref, len_ref, o5_ref, r4_ref = refs
        j = pl.program_id(0)
        ci = lax.broadcasted_iota(jnp.int32, (128, 128), 1)
        pts = []
        for bcl in range(NWQ):
            s = jnp.zeros((128, 128), jnp.float32)
            for r in range(TG // 2):
                xr = lin_ref[pl.ds(bcl * 128, 128), r, :]
                o5_ref[pl.ds(2 * r, 2), :, bcl, :, :] = (
                    jnp.transpose(xr, (1, 0)).reshape(2, 8, 8, 128))
                tmask = (j * TG + 2 * r + ci // DIM) < SEQ
                s = s + jnp.where(tmask, xr, 0.0)
            part = s[:, :DIM] + s[:, DIM:]
            pts.append(jnp.transpose(part, (1, 0)).reshape(8, 1, 8, 128))
        pt = jnp.concatenate(pts, axis=1)

        @pl.when(j == 0)
        def _():
            r4_ref[...] = pt

        @pl.when(j > 0)
        def _():
            r4_ref[...] = r4_ref[...] + pt

        @pl.when(j == NTG - 1)
        def _():
            for bcl in range(NWQ):
                recip = 1.0 / len_ref[pl.ds(bcl * 128, 128)].astype(
                    jnp.float32)
                r4_ref[:, bcl, :, :] = (r4_ref[:, bcl, :, :]
                                        * recip.reshape(1, 1, 128))

    return body


def _run_quarter_sc(h, x2, emb_weight):
    mesh = plsc.VectorSubcoreMesh(core_axis_name="c", subcore_axis_name="s")
    sck = pl.kernel(
        _make_sc_body(h),
        mesh=mesh,
        compiler_params=pltpu.CompilerParams(
            needs_layout_passes=False, use_tc_tiling_on_sc=False),
        out_type=(
            jax.ShapeDtypeStruct((BQ, SEQ_PAD * DIM // 128, 128),
                                 jnp.float32),
        ),
        scratch_types=[
            pltpu.VMEM((NBUF, SUB, ROWS_PER_SUB), jnp.int32),
            pltpu.VMEM((NBUF, SUB, ROWS_PER_SUB, DIM), jnp.float32),
            [pltpu.SemaphoreType.DMA] * NBUF,
            [pltpu.SemaphoreType.DMA] * NBUF,
        ],
    )
    (lin,) = sck(x2, emb_weight)
    return lin


def _run_quarter_tc(h, lin, len_h, o5_prev, r4_prev):
    aliased = o5_prev is not None
    out_shape = (
        jax.ShapeDtypeStruct((SEQ, 8, NW, 8, 128), jnp.float32),
        jax.ShapeDtypeStruct((8, NW, 8, 128), jnp.float32),
    )
    data_specs = [
        pl.BlockSpec((BQ, TG * DIM // 128, 128), lambda j: (0, j, 0)),
        pl.BlockSpec((BQ,), lambda j: (0,)),
    ]
    out_specs = [
        pl.BlockSpec((TG, 8, NWQ, 8, 128), lambda j, h=h: (j, 0, h, 0, 0)),
        pl.BlockSpec((8, NWQ, 8, 128), lambda j, h=h: (0, h, 0, 0)),
    ]
    if aliased:
        in_specs = [pl.BlockSpec(memory_space=pl.ANY),
                    pl.BlockSpec(memory_space=pl.ANY)] + data_specs
        args = (o5_prev, r4_prev, lin, len_h)
        aliases = {0: 0, 1: 1}
    else:
        in_specs = data_specs
        args = (lin, len_h)
        aliases = {}
    return pl.pallas_call(
        _make_tc_body(h, aliased),
        grid=(NTG,),
        in_specs=in_specs,
        out_specs=out_specs,
        out_shape=out_shape,
        input_output_aliases=aliases,
    )(*args)


@jax.jit
def _run(x2, x_len, emb_weight):
    o5 = r4 = None
    lens = x_len.reshape(NH, BQ)
    for h in range(NH):
        lin = _run_quarter_sc(h, x2, emb_weight)
        o5, r4 = _run_quarter_tc(h, lin, lens[h], o5, r4)
    return o5, r4


def kernel(x, x_len, emb_weight):
    perm = jnp.concatenate([jnp.arange(0, SEQ, 2), jnp.arange(1, SEQ, 2)])
    x2 = x[:, perm].astype(jnp.int32).reshape(XROWS, ROWS_PER_SUB)
    o5, r4 = _run(x2, x_len.astype(jnp.int32), emb_weight)
    out = o5.transpose((2, 4, 0, 1, 3)).reshape(BATCH, SEQ, DIM)
    ret = r4.transpose((1, 3, 0, 2)).reshape(BATCH, DIM)
    return (ret, out)
